# baseline (speedup 1.0000x reference)
import functools

import jax
import jax.numpy as jnp
from jax import lax
from jax.experimental import pallas as pl
from jax.experimental.pallas import tpu as pltpu
from jax.experimental.pallas import tpu_sc as plsc

_HIDDEN = 64
_EPS = 1e-12
_NC = 2
_NS = 16
_NW = _NC * _NS


def _ln_body(rpw, chunk, idx_hbm, table_hbm, gamma_hbm, beta_hbm, out_hbm,
             idx_v, rows_v, gb_v, sem):
    wid = lax.axis_index("s") * _NC + lax.axis_index("c")
    base = wid * rpw
    nchunk = rpw // chunk

    pltpu.sync_copy(gamma_hbm, gb_v.at[0])
    pltpu.sync_copy(beta_hbm, gb_v.at[1])
    g = [gb_v[0, pl.ds(16 * h, 16)] for h in range(4)]
    b = [gb_v[1, pl.ds(16 * h, 16)] for h in range(4)]
    lanes = lax.iota(jnp.int32, 16)
    lanes64 = lanes * 64

    def chunk_body(c, _):
        start = base + c * chunk
        pltpu.sync_copy(idx_hbm.at[pl.ds(start, chunk)], idx_v)
        pltpu.async_copy(table_hbm.at[idx_v], rows_v, sem).wait()

        def group_body(gi, _):
            r0 = gi * 16
            rowv = lanes + r0
            acc = jnp.zeros((16,), jnp.float32)
            acq = jnp.zeros((16,), jnp.float32)
            for h in range(64):
                xh = plsc.load_gather(rows_v, [rowv, jnp.full((16,), h, jnp.int32)])
                acc = acc + xh
                acq = acq + xh * xh
            mean = acc * (1.0 / 64.0)
            v = acq * (1.0 / 64.0) - mean * mean + _EPS
            i = lax.bitcast_convert_type(v, jnp.int32)
            i = jnp.int32(0x5F3759DF) - lax.shift_right_logical(i, 1)
            y = lax.bitcast_convert_type(i, jnp.float32)
            hv = 0.5 * v
            y = y * (1.5 - hv * y * y)
            y = y * (1.5 - hv * y * y)
            y = y * (1.5 - hv * y * y)
            for ri in range(16):
                r = r0 + ri
                m = mean[ri]
                q = y[ri]
                for h in range(4):
                    sl = pl.ds(16 * h, 16)
                    rows_v[r, sl] = (rows_v[r, sl] - m) * q * g[h] + b[h]
            return 0

        lax.fori_loop(0, chunk // 16, group_body, 0)
        pltpu.sync_copy(rows_v, out_hbm.at[pl.ds(start, chunk)])
        return 0

    lax.fori_loop(0, nchunk, chunk_body, 0)


def _make_call(rows, chunk):
    rpw = rows // _NW
    mesh = plsc.VectorSubcoreMesh(core_axis_name="c", subcore_axis_name="s")
    return pl.kernel(
        functools.partial(_ln_body, rpw, chunk),
        out_type=jax.ShapeDtypeStruct((rows, _HIDDEN), jnp.float32),
        mesh=mesh,
        scratch_types=[
            pltpu.VMEM((chunk,), jnp.int32),
            pltpu.VMEM((chunk, _HIDDEN), jnp.float32),
            pltpu.VMEM((2, _HIDDEN), jnp.float32),
            pltpu.SemaphoreType.DMA,
        ],
        compiler_params=pltpu.CompilerParams(use_tc_tiling_on_sc=False,
                                             needs_layout_passes=False),
    )


@jax.jit
def kernel(idiom_ids, table, gamma, beta):
    bsz, seq = idiom_ids.shape
    rows = bsz * seq
    idx = idiom_ids.reshape(rows).astype(jnp.int32)
    out = _make_call(rows, 1024)(idx, table, gamma, beta)
    return out.reshape(bsz, seq, _HIDDEN)


# speedup vs baseline: 1.7760x; 1.7760x over previous
import functools

import jax
import jax.numpy as jnp
from jax import lax
from jax.experimental import pallas as pl
from jax.experimental.pallas import tpu as pltpu
from jax.experimental.pallas import tpu_sc as plsc

_HIDDEN = 64
_EPS = 1e-12
_NC = 2
_NS = 16
_NW = _NC * _NS
_UNROLL = 4


def _ln_body(rpw, chunk, idx_hbm, table_hbm, gamma_hbm, beta_hbm, out_hbm,
             idx_all, rows0, rows1, gb_v, gsem, wsem):
    wid = lax.axis_index("s") * _NC + lax.axis_index("c")
    base = wid * rpw
    nchunk = rpw // chunk
    rows = (rows0, rows1)

    pltpu.sync_copy(idx_hbm.at[pl.ds(base, rpw)], idx_all)
    pltpu.sync_copy(gamma_hbm, gb_v.at[0])
    pltpu.sync_copy(beta_hbm, gb_v.at[1])
    g = [gb_v[0, pl.ds(16 * h, 16)] for h in range(4)]
    b = [gb_v[1, pl.ds(16 * h, 16)] for h in range(4)]
    lanes = lax.iota(jnp.int32, 16)
    perms = [lax.bitwise_xor(lanes, jnp.int32(1 << p)) for p in range(4)]
    dnums = lax.GatherDimensionNumbers(
        offset_dims=(), collapsed_slice_dims=(0,), start_index_map=(0,))

    def _shuf(x, p):
        return lax.gather(x, p.reshape(16, 1), dnums, (1,),
                          indices_are_sorted=False, unique_indices=True,
                          mode=lax.GatherScatterMode.PROMISE_IN_BOUNDS)

    def _gather(c, buf, sem):
        src = table_hbm.at[idx_all.at[pl.ds(c * chunk, chunk)]]
        pltpu.async_copy(src, buf, sem)

    def _row(rv, r):
        x = [rv[r, pl.ds(16 * h, 16)] for h in range(4)]
        s = (x[0] + x[1]) + (x[2] + x[3])
        q = (x[0] * x[0] + x[1] * x[1]) + (x[2] * x[2] + x[3] * x[3])
        for p in perms:
            s = s + _shuf(s, p)
            q = q + _shuf(q, p)
        mean = s * (1.0 / 64.0)
        v = q * (1.0 / 64.0) - mean * mean + _EPS
        i = lax.bitcast_convert_type(v, jnp.int32)
        i = jnp.int32(0x5F3759DF) - lax.shift_right_logical(i, 1)
        y = lax.bitcast_convert_type(i, jnp.float32)
        hv = 0.5 * v
        y = y * (1.5 - hv * y * y)
        y = y * (1.5 - hv * y * y)
        y = y * (1.5 - hv * y * y)
        for h in range(4):
            rv[r, pl.ds(16 * h, 16)] = (x[h] - mean) * y * g[h] + b[h]

    _gather(0, rows[0], gsem.at[0])

    def step_body(st, _):
        for bi in range(2):
            c = st * 2 + bi
            cur, oth = rows[bi], rows[1 - bi]

            @pl.when(c >= 1)
            def _():
                pltpu.make_async_copy(
                    oth, out_hbm.at[pl.ds(0, chunk)], wsem.at[1 - bi]).wait()

            @pl.when(c + 1 < nchunk)
            def _():
                _gather(c + 1, oth, gsem.at[1 - bi])

            pltpu.make_async_copy(
                table_hbm.at[idx_all.at[pl.ds(0, chunk)]], cur,
                gsem.at[bi]).wait()

            def row_block(t, _):
                r0 = t * _UNROLL
                for u in range(_UNROLL):
                    _row(cur, r0 + u)
                return 0

            lax.fori_loop(0, chunk // _UNROLL, row_block, 0)
            pltpu.async_copy(cur, out_hbm.at[pl.ds(base + c * chunk, chunk)],
                             wsem.at[bi])
        return 0

    lax.fori_loop(0, nchunk // 2, step_body, 0)
    pltpu.make_async_copy(rows[1], out_hbm.at[pl.ds(0, chunk)],
                          wsem.at[(nchunk - 1) % 2]).wait()


def _make_call(rows, chunk):
    rpw = rows // _NW
    mesh = plsc.VectorSubcoreMesh(core_axis_name="c", subcore_axis_name="s")
    return pl.kernel(
        functools.partial(_ln_body, rpw, chunk),
        out_type=jax.ShapeDtypeStruct((rows, _HIDDEN), jnp.float32),
        mesh=mesh,
        scratch_types=[
            pltpu.VMEM((rpw,), jnp.int32),
            pltpu.VMEM((chunk, _HIDDEN), jnp.float32),
            pltpu.VMEM((chunk, _HIDDEN), jnp.float32),
            pltpu.VMEM((2, _HIDDEN), jnp.float32),
            pltpu.SemaphoreType.DMA((2,)),
            pltpu.SemaphoreType.DMA((2,)),
        ],
        compiler_params=pltpu.CompilerParams(use_tc_tiling_on_sc=False,
                                             needs_layout_passes=False),
    )


@jax.jit
def kernel(idiom_ids, table, gamma, beta):
    bsz, seq = idiom_ids.shape
    rows = bsz * seq
    idx = idiom_ids.reshape(rows).astype(jnp.int32)
    out = _make_call(rows, 640)(idx, table, gamma, beta)
    return out.reshape(bsz, seq, _HIDDEN)
